# baseline (speedup 1.0000x reference)
import functools

import jax
import jax.numpy as jnp
from jax import lax
from jax.experimental import pallas as pl
from jax.experimental.pallas import tpu as pltpu
from jax.experimental.pallas import tpu_sc as plsc

D = 64
NB = 4096
T = 200
SPLITS = ((0, 128), (128, 72))
CHUNK_XR = 2

_info = plsc.get_sparse_core_info()
NC, NS = _info.num_cores, _info.num_subcores
NW = NC * NS
XR_PER_W = NB // NW
CHUNKS = XR_PER_W // CHUNK_XR

_mesh = plsc.VectorSubcoreMesh(core_axis_name="c", subcore_axis_name="s")


@functools.partial(
    pl.kernel,
    mesh=_mesh,
    compiler_params=pltpu.CompilerParams(use_tc_tiling_on_sc=False),
    out_type=jax.ShapeDtypeStruct((NB, T, D), jnp.float32),
    scratch_types=[
        pltpu.VMEM((XR_PER_W, T), jnp.int32),
        pltpu.VMEM((2, CHUNK_XR, T, D), jnp.float32),
        pltpu.SemaphoreType.DMA((2,)),
        pltpu.SemaphoreType.DMA((2,)),
    ],
)
def _emb_lookup(w_hbm, x_hbm, out_hbm, idx_v, rows_v, gsem, wsem):
    wid = lax.axis_index("s") * NC + lax.axis_index("c")
    xr0 = wid * XR_PER_W
    pltpu.sync_copy(x_hbm.at[pl.ds(xr0, XR_PER_W)], idx_v)

    def gather_copies(g, b):
        r0 = g * CHUNK_XR
        return [
            pltpu.make_async_copy(
                w_hbm.at[idx_v.at[r0 + r, pl.ds(off, n)]],
                rows_v.at[b, r, pl.ds(off, n)],
                gsem.at[b])
            for r in range(CHUNK_XR)
            for (off, n) in SPLITS
        ]

    def wb_copy(g, b):
        return pltpu.make_async_copy(
            rows_v.at[b],
            out_hbm.at[pl.ds(xr0 + g * CHUNK_XR, CHUNK_XR)],
            wsem.at[b])

    for cp in gather_copies(0, 0):
        cp.start()

    def chunk(g, carry):
        b = lax.rem(g, 2)
        nb = 1 - b
        for cp in gather_copies(g, b):
            cp.wait()
        wb_copy(g, b).start()

        @pl.when(g + 1 < CHUNKS)
        def _():
            @pl.when(g >= 1)
            def _():
                wb_copy(g - 1, nb).wait()
            for cp in gather_copies(g + 1, nb):
                cp.start()

        return carry

    lax.fori_loop(0, CHUNKS, chunk, 0)
    wb_copy(CHUNKS - 2, lax.rem(CHUNKS - 2, 2)).wait()
    wb_copy(CHUNKS - 1, lax.rem(CHUNKS - 1, 2)).wait()


def kernel(X, W):
    return _emb_lookup(W, X.astype(jnp.int32))
